# Initial kernel scaffold; baseline (speedup 1.0000x reference)
import jax
import jax.numpy as jnp
from jax.experimental import pallas as pl


def kernel(h, e, edge_index, params):
    raise NotImplementedError("write your pallas kernel here")


# speedup vs baseline: 1.8846x; 1.8846x over previous
import functools

import jax
import jax.numpy as jnp
from jax import lax
from jax.experimental import pallas as pl
from jax.experimental.pallas import tpu as pltpu
from jax.experimental.pallas import tpu_sc as plsc

DIM = 128
EDIM = 16
NB = 1000
TB = 512
BLK = 128
NW = 32


def _node_tables_body(h_ref, wnode_ref, nbias_ref, t1_ref, t2_ref):
    h = h_ref[...]
    proj = jnp.dot(h, wnode_ref[...], preferred_element_type=jnp.float32)
    hs = proj[:, :DIM] + nbias_ref[0:1, :DIM]
    hm = proj[:, DIM:2 * DIM] + nbias_ref[0:1, DIM:2 * DIM]
    hd = proj[:, 2 * DIM:] + nbias_ref[0:1, 2 * DIM:]
    t1_ref[...] = jnp.concatenate([hs, hm], axis=1)
    t2_ref[...] = jnp.concatenate([hd, hm], axis=1)


def _edge_body(gs_ref, gd_ref, e_ref, egw_ref, w1e_ref, w1s_ref, w1d_ref,
               w2_ref, evec_ref, m_ref, enew_ref):
    e = e_ref[...]
    eg = jnp.dot(e, egw_ref[...], preferred_element_type=jnp.float32)
    eg = eg + evec_ref[0:1, :DIM]
    gl = gs_ref[:, :DIM] + gd_ref[:, :DIM] + eg
    gate = jax.nn.sigmoid(gl)
    m_ref[...] = gate * gs_ref[:, DIM:]

    hid = jnp.dot(e, w1e_ref[...], preferred_element_type=jnp.float32)
    hid = hid + jnp.dot(gs_ref[:, DIM:], w1s_ref[...],
                        preferred_element_type=jnp.float32)
    hid = hid + jnp.dot(gd_ref[:, DIM:], w1d_ref[...],
                        preferred_element_type=jnp.float32)
    hid = hid + evec_ref[0:1, DIM:DIM + EDIM]
    act = hid * jax.nn.sigmoid(hid)
    eu = jnp.dot(act, w2_ref[...], preferred_element_type=jnp.float32)
    eu = eu + evec_ref[0:1, DIM + EDIM:DIM + 2 * EDIM]
    r = e + eu
    mu = jnp.mean(r, axis=1, keepdims=True)
    var = jnp.mean((r - mu) ** 2, axis=1, keepdims=True)
    g = evec_ref[0:1, DIM + 2 * EDIM:DIM + 3 * EDIM]
    b = evec_ref[0:1, DIM + 3 * EDIM:DIM + 4 * EDIM]
    enew_ref[...] = (r - mu) * lax.rsqrt(var + 1e-5) * g + b


def _node_update_body(h_ref, agg_ref, w1_ref, w2_ref, vec_ref, out_ref):
    h = h_ref[...]
    agg = agg_ref[0] + agg_ref[1]
    nu_in = jnp.concatenate([h, agg], axis=1)
    t = jnp.dot(nu_in, w1_ref[...], preferred_element_type=jnp.float32)
    t = t + vec_ref[0:1, :DIM]
    t = t * jax.nn.sigmoid(t)
    nu = jnp.dot(t, w2_ref[...], preferred_element_type=jnp.float32)
    nu = nu + vec_ref[0:1, DIM:2 * DIM]
    r = h + nu
    mu = jnp.mean(r, axis=1, keepdims=True)
    var = jnp.mean((r - mu) ** 2, axis=1, keepdims=True)
    g = vec_ref[0:1, 2 * DIM:3 * DIM]
    b = vec_ref[0:1, 3 * DIM:]
    out_ref[...] = (r - mu) * lax.rsqrt(var + 1e-5) * g + b


def _block_range(wid, nblk):
    nfull = nblk // NW
    extra = nblk - nfull * NW
    start = wid * nfull + jnp.minimum(wid, extra)
    cnt = nfull + jnp.where(wid < extra, 1, 0)
    return start, cnt


def _sc_gather_body(src_hbm, dst_hbm, t1_hbm, t2_hbm,
                    gs_hbm, gd_hbm,
                    idx_s, idx_d, r1, r2, sem, *, nblk):
    wid = lax.axis_index("s") * 2 + lax.axis_index("c")
    start, cnt = _block_range(wid, nblk)

    def body(k, carry):
        base = (start + k) * BLK
        pltpu.sync_copy(src_hbm.at[pl.ds(base, BLK)], idx_s)
        pltpu.sync_copy(dst_hbm.at[pl.ds(base, BLK)], idx_d)
        cp1 = pltpu.async_copy(t1_hbm.at[idx_s], r1, sem)
        cp2 = pltpu.async_copy(t2_hbm.at[idx_d], r2, sem)
        cp1.wait()
        cp2.wait()
        pltpu.sync_copy(r1, gs_hbm.at[pl.ds(base, BLK)])
        pltpu.sync_copy(r2, gd_hbm.at[pl.ds(base, BLK)])
        return carry

    lax.fori_loop(0, cnt, body, 0)


def _sc_scatter_body(dst_hbm, m_hbm, z_hbm, out_hbm,
                     agg_s, idx, mrows, sem, *, nblk, n_nodes):
    cid = lax.axis_index("c")
    sid = lax.axis_index("s")
    wid = sid * 2 + cid
    rows_per = ((n_nodes // 16 + 7) // 8) * 8
    tail = n_nodes - 15 * rows_per

    @pl.when(sid < 15)
    def _():
        pltpu.sync_copy(z_hbm, agg_s.at[pl.ds(sid * rows_per, rows_per)])

    @pl.when(sid == 15)
    def _():
        pltpu.sync_copy(z_hbm.at[pl.ds(0, tail)],
                        agg_s.at[pl.ds(15 * rows_per, tail)])

    plsc.subcore_barrier()

    start, cnt = _block_range(wid, nblk)

    def body(k, carry):
        base = (start + k) * BLK
        pltpu.sync_copy(dst_hbm.at[pl.ds(base, BLK)], idx)
        pltpu.sync_copy(m_hbm.at[pl.ds(base, BLK)], mrows)
        pltpu.sync_copy(mrows, agg_s.at[idx], add=True)
        return carry

    lax.fori_loop(0, cnt, body, 0)
    plsc.subcore_barrier()

    @pl.when(sid < 15)
    def _():
        pltpu.sync_copy(agg_s.at[pl.ds(sid * rows_per, rows_per)],
                        out_hbm.at[cid, pl.ds(sid * rows_per, rows_per)])

    @pl.when(sid == 15)
    def _():
        pltpu.sync_copy(agg_s.at[pl.ds(15 * rows_per, tail)],
                        out_hbm.at[cid, pl.ds(15 * rows_per, tail)])


def kernel(h, e, edge_index, params):
    n_nodes, _ = h.shape
    n_edges = e.shape[0]
    nblk = n_edges // BLK

    src = edge_index[0].astype(jnp.int32)
    dst = edge_index[1].astype(jnp.int32)

    wnode = jnp.concatenate(
        [params['src_W'].T, params['msg_W'].T, params['dst_W'].T], axis=1)
    nbias = jnp.concatenate(
        [params['src_b'], params['msg_b'], params['dst_b']])[None, :]
    w1 = params['eu_W1']
    egw = params['eg_W'].T
    w1e = w1[:, :EDIM].T
    w1s = w1[:, EDIM:EDIM + DIM].T
    w1d = w1[:, EDIM + DIM:].T
    w2 = params['eu_W2'].T
    evec = jnp.concatenate(
        [params['eg_b'], params['eu_b1'], params['eu_b2'],
         params['en_g'], params['en_b']])[None, :]
    nw1 = params['nu_W1'].T
    nw2 = params['nu_W2'].T
    nvec = jnp.concatenate(
        [params['nu_b1'], params['nu_b2'],
         params['nn_g'], params['nn_b']])[None, :]

    f32 = jnp.float32

    t1, t2 = pl.pallas_call(
        _node_tables_body,
        grid=(n_nodes // NB,),
        in_specs=[
            pl.BlockSpec((NB, DIM), lambda i: (i, 0)),
            pl.BlockSpec((DIM, 3 * DIM), lambda i: (0, 0)),
            pl.BlockSpec((1, 3 * DIM), lambda i: (0, 0)),
        ],
        out_specs=[
            pl.BlockSpec((NB, 2 * DIM), lambda i: (i, 0)),
            pl.BlockSpec((NB, 2 * DIM), lambda i: (i, 0)),
        ],
        out_shape=[
            jax.ShapeDtypeStruct((n_nodes, 2 * DIM), f32),
            jax.ShapeDtypeStruct((n_nodes, 2 * DIM), f32),
        ],
    )(h, wnode, nbias)

    mesh = plsc.VectorSubcoreMesh(core_axis_name="c", subcore_axis_name="s")
    gather = pl.kernel(
        functools.partial(_sc_gather_body, nblk=nblk),
        mesh=mesh,
        out_type=[
            jax.ShapeDtypeStruct((n_edges, 2 * DIM), f32),
            jax.ShapeDtypeStruct((n_edges, 2 * DIM), f32),
        ],
        scratch_types=[
            pltpu.VMEM((BLK,), jnp.int32),
            pltpu.VMEM((BLK,), jnp.int32),
            pltpu.VMEM((BLK, 2 * DIM), f32),
            pltpu.VMEM((BLK, 2 * DIM), f32),
            pltpu.SemaphoreType.DMA,
        ],
    )
    gs, gd = gather(src, dst, t1, t2)

    m, e_new = pl.pallas_call(
        _edge_body,
        grid=(n_edges // TB,),
        in_specs=[
            pl.BlockSpec((TB, 2 * DIM), lambda i: (i, 0)),
            pl.BlockSpec((TB, 2 * DIM), lambda i: (i, 0)),
            pl.BlockSpec((TB, EDIM), lambda i: (i, 0)),
            pl.BlockSpec((EDIM, DIM), lambda i: (0, 0)),
            pl.BlockSpec((EDIM, EDIM), lambda i: (0, 0)),
            pl.BlockSpec((DIM, EDIM), lambda i: (0, 0)),
            pl.BlockSpec((DIM, EDIM), lambda i: (0, 0)),
            pl.BlockSpec((EDIM, EDIM), lambda i: (0, 0)),
            pl.BlockSpec((1, DIM + 4 * EDIM), lambda i: (0, 0)),
        ],
        out_specs=[
            pl.BlockSpec((TB, DIM), lambda i: (i, 0)),
            pl.BlockSpec((TB, EDIM), lambda i: (i, 0)),
        ],
        out_shape=[
            jax.ShapeDtypeStruct((n_edges, DIM), f32),
            jax.ShapeDtypeStruct((n_edges, EDIM), f32),
        ],
    )(gs, gd, e, egw, w1e, w1s, w1d, w2, evec)

    z = jnp.zeros((((n_nodes // 16 + 7) // 8) * 8, DIM), f32)
    scatter = pl.kernel(
        functools.partial(_sc_scatter_body, nblk=nblk, n_nodes=n_nodes),
        mesh=mesh,
        out_type=jax.ShapeDtypeStruct((2, n_nodes, DIM), f32),
        scratch_types=[
            pltpu.VMEM_SHARED((n_nodes, DIM), f32),
            pltpu.VMEM((BLK,), jnp.int32),
            pltpu.VMEM((BLK, DIM), f32),
            pltpu.SemaphoreType.DMA,
        ],
    )
    agg_parts = scatter(dst, m, z)

    h_new = pl.pallas_call(
        _node_update_body,
        grid=(n_nodes // NB,),
        in_specs=[
            pl.BlockSpec((NB, DIM), lambda i: (i, 0)),
            pl.BlockSpec((2, NB, DIM), lambda i: (0, i, 0)),
            pl.BlockSpec((2 * DIM, DIM), lambda i: (0, 0)),
            pl.BlockSpec((DIM, DIM), lambda i: (0, 0)),
            pl.BlockSpec((1, 4 * DIM), lambda i: (0, 0)),
        ],
        out_specs=pl.BlockSpec((NB, DIM), lambda i: (i, 0)),
        out_shape=jax.ShapeDtypeStruct((n_nodes, DIM), f32),
    )(h, agg_parts, nw1, nw2, nvec)

    return (h_new, e_new)
